# Initial kernel scaffold; baseline (speedup 1.0000x reference)
import functools

import jax
import jax.numpy as jnp
from jax import lax
from jax.experimental import pallas as pl
from jax.experimental.pallas import tpu as pltpu
from jax.experimental.pallas import tpu_sc as plsc

_N = 1000000
_CLS = 20
_NC = 2
_NS = 16
_NW = _NC * _NS
_ROWS_W = _N // _NW
_ROWS_W_PAD = ((_ROWS_W + 15) // 16) * 16
_P1_CHUNK = 2000
_L = 16
_GB = 128

_mesh = plsc.VectorSubcoreMesh(
    core_axis_name="c", subcore_axis_name="s", num_cores=_NC, num_subcores=_NS
)


def _wid():
    return lax.axis_index("c") * _NS + lax.axis_index("s")


def _copy_body(p_ref, out_ref, cls_ref):
    x = p_ref[...]
    out_ref[...] = x
    iota = lax.broadcasted_iota(jnp.float32, x.shape, 1)
    cls = jnp.sum(x * iota, axis=1).astype(jnp.int32)
    cls_ref[0] = cls.reshape(cls_ref.shape[1:])


def _tc_copy(primary):
    blk = 8000
    nblk = _N // blk
    return pl.pallas_call(
        _copy_body,
        grid=(nblk,),
        in_specs=[pl.BlockSpec((blk, _CLS), lambda i: (i, 0))],
        out_specs=[
            pl.BlockSpec((blk, _CLS), lambda i: (i, 0)),
            pl.BlockSpec((1, blk // 128, 128), lambda i: (i, 0, 0)),
        ],
        out_shape=[
            jax.ShapeDtypeStruct((_N, _CLS), jnp.float32),
            jax.ShapeDtypeStruct((nblk, blk // 128, 128), jnp.int32),
        ],
    )(primary)


def _p1_body(pos_hbm, val_hbm, bits_hbm, bits_v, pos_v, val_v, sem):
    npairs = pos_hbm.shape[0]
    nchunk = npairs // _P1_CHUNK
    wid = _wid()
    base = wid * _ROWS_W

    def _zero(i, _):
        bits_v[pl.ds(i * _L, _L)] = jnp.zeros((_L,), jnp.int32)
        return 0

    lax.fori_loop(0, _ROWS_W_PAD // _L, _zero, 0)

    def _chunk(k, _):
        pltpu.async_copy(pos_hbm.at[pl.ds(k * _P1_CHUNK, _P1_CHUNK)], pos_v, sem).wait()
        pltpu.async_copy(val_hbm.at[pl.ds(k * _P1_CHUNK, _P1_CHUNK)], val_v, sem).wait()

        def _group(g, _):
            pos = pos_v[pl.ds(g * _L, _L)]
            val = val_v[pl.ds(g * _L, _L)]
            rel = pos - base
            m = (rel >= 0) & (rel < _ROWS_W)
            any_in = jnp.max(jnp.where(m, 1, 0)) > 0

            @pl.when(any_in)
            def _():
                srel = jnp.clip(rel, 0, _ROWS_W - 1)
                bitv = jnp.int32(1) << val

                def _cond(pending):
                    return jnp.max(jnp.where(pending, 1, 0)) > 0

                def _rmw(pending):
                    old = plsc.load_gather(bits_v, [srel], mask=pending)
                    plsc.store_scatter(bits_v, [srel], old | bitv, mask=pending)
                    cur = plsc.load_gather(bits_v, [srel], mask=pending)
                    return pending & ((cur & bitv) != bitv)

                lax.while_loop(_cond, _rmw, m)

            return 0

        lax.fori_loop(0, _P1_CHUNK // _L, _group, 0)
        return 0

    lax.fori_loop(0, nchunk, _chunk, 0)
    pltpu.async_copy(bits_v.at[pl.ds(0, _ROWS_W)], bits_hbm.at[wid], sem).wait()


def _sc_phase1(pos1, val1):
    return pl.kernel(
        _p1_body,
        out_type=jax.ShapeDtypeStruct((_NW, _ROWS_W), jnp.int32),
        mesh=_mesh,
        scratch_types=[
            pltpu.VMEM((_ROWS_W_PAD,), jnp.int32),
            pltpu.VMEM((_P1_CHUNK,), jnp.int32),
            pltpu.VMEM((_P1_CHUNK,), jnp.int32),
            pltpu.SemaphoreType.DMA,
        ],
    )(pos1, val1)


def _p2_body(out_ref, bits_hbm, clsp_hbm, pos2_hbm, w_hbm, b_hbm,
             p2_v, bits_b, cls_b, a1, v1, a2, v2, w_v, b_v, t_v,
             sem1, sem2, sem3, sem4):
    nrow = p2_v.shape[0]
    wid = _wid()
    pltpu.async_copy(pos2_hbm.at[wid], p2_v, sem1).wait()
    pltpu.async_copy(w_hbm, w_v, sem1).wait()
    pltpu.async_copy(b_hbm, b_v, sem1).wait()

    def _lut(c, _):
        def _scan(j, carry):
            best, bi = carry
            lg = w_v[c, j] + b_v[j]
            take = lg > best
            return jnp.where(take, lg, best), jnp.where(take, j, bi)

        _, bi = lax.fori_loop(1, _CLS, _scan, (w_v[c, 0] + b_v[0], 0))
        t_v[c] = bi
        return 0

    lax.fori_loop(0, _CLS, _lut, 0)

    gathers = []
    for j in range(nrow):
        gathers.append(pltpu.async_copy(bits_hbm.at[p2_v.at[j]], bits_b.at[j], sem1))
        gathers.append(pltpu.async_copy(clsp_hbm.at[p2_v.at[j]], cls_b.at[j], sem2))
    for g in gathers:
        g.wait()

    def _row(j, _):
        for l in range(_GB // _L):
            sl = pl.ds(l * _L, _L)
            pos = p2_v[j, sl]
            bits = bits_b[j, sl]
            c_old = cls_b[j, sl]
            is0 = bits == 0
            one_hot = (bits & (bits - 1)) == 0
            f = bits.astype(jnp.float32)
            expo = (plsc.bitcast(f, jnp.int32) >> 23) - 127
            c_one = jnp.clip(jnp.where(is0, c_old, expo), 0, _CLS - 1)
            cls = plsc.load_gather(t_v, [c_one])

            multi = ~one_hot
            any_multi = jnp.max(jnp.where(multi, 1, 0)) > 0

            def _hard():
                def _outer(j2, carry):
                    best, bi = carry

                    def _inner(v, acc):
                        on = ((bits >> v) & 1) == 1
                        return acc + jnp.where(on, w_v[v, j2], 0.0)

                    lg = lax.fori_loop(0, _CLS, _inner, jnp.zeros((_L,), jnp.float32))
                    lg = lg + b_v[j2]
                    take = lg > best
                    return jnp.where(take, lg, best), jnp.where(take, j2, bi)

                init = (jnp.full((_L,), -jnp.inf, jnp.float32),
                        jnp.zeros((_L,), jnp.int32))
                _, bi = lax.fori_loop(0, _CLS, _outer, init)
                return jnp.where(multi, bi, cls)

            cls = lax.cond(any_multi, _hard, lambda: cls)

            a1[j, sl] = pos * _CLS + c_old
            v1[j, sl] = jnp.where(cls == c_old, 1.0, 0.0)
            a2[j, sl] = pos * _CLS + cls
            v2[j, sl] = jnp.ones((_L,), jnp.float32)
        return 0

    lax.fori_loop(0, nrow, _row, 0)

    scatters = []
    for j in range(nrow):
        scatters.append(pltpu.async_copy(v1.at[j], out_ref.at[a1.at[j]], sem3))
        scatters.append(pltpu.async_copy(v2.at[j], out_ref.at[a2.at[j]], sem4))
    for s in scatters:
        s.wait()


def _sc_phase2(out_flat_ref, bits, clsp, pos2, w, b_pad, nrow):
    pl.kernel(
        _p2_body,
        out_type=(),
        mesh=_mesh,
        scratch_types=[
            pltpu.VMEM((nrow, _GB), jnp.int32),
            pltpu.VMEM((nrow, _GB), jnp.int32),
            pltpu.VMEM((nrow, _GB), jnp.int32),
            pltpu.VMEM((nrow, _GB), jnp.int32),
            pltpu.VMEM((nrow, _GB), jnp.float32),
            pltpu.VMEM((nrow, _GB), jnp.int32),
            pltpu.VMEM((nrow, _GB), jnp.float32),
            pltpu.VMEM((_CLS, _CLS), jnp.float32),
            pltpu.VMEM((32,), jnp.float32),
            pltpu.VMEM((32,), jnp.int32),
            pltpu.SemaphoreType.DMA,
            pltpu.SemaphoreType.DMA,
            pltpu.SemaphoreType.DMA,
            pltpu.SemaphoreType.DMA,
        ],
    )(out_flat_ref, bits, clsp, pos2, w, b_pad)


def kernel(primary, W, b, positions1, values1, positions2):
    p1 = positions1.shape[0]
    p2 = positions2.shape[0]

    pad1 = (-p1) % _P1_CHUNK
    if pad1:
        positions1 = jnp.concatenate(
            [positions1, jnp.full((pad1,), -1, positions1.dtype)])
        values1 = jnp.concatenate([values1, jnp.zeros((pad1,), values1.dtype)])

    pad2 = (-p2) % (_NW * _GB)
    if pad2:
        positions2 = jnp.concatenate(
            [positions2, jnp.broadcast_to(positions2[:1], (pad2,))])
    nrow = positions2.shape[0] // (_NW * _GB)
    pos2 = positions2.reshape(_NW, nrow, _GB).astype(jnp.int32)

    b_pad = jnp.concatenate([b, jnp.zeros((32 - _CLS,), b.dtype)])

    out0, cls3 = _tc_copy(primary)
    clsp = cls3.reshape(_N)
    bits = _sc_phase1(positions1.astype(jnp.int32),
                      values1.astype(jnp.int32)).reshape(_N)

    oref = jax.new_ref(out0.reshape(_N * _CLS))
    _sc_phase2(oref, bits, clsp, pos2, W, b_pad, nrow)
    return oref[...].reshape(_N, _CLS)


# speedup vs baseline: 7.0668x; 7.0668x over previous
import functools

import jax
import jax.numpy as jnp
from jax import lax
from jax.experimental import pallas as pl
from jax.experimental.pallas import tpu as pltpu
from jax.experimental.pallas import tpu_sc as plsc

_N = 1000000
_CLS = 20
_NC = 2
_NS = 16
_NW = _NC * _NS
_ROWS_W = _N // _NW
_ROWS_W_PAD = ((_ROWS_W + 15) // 16) * 16
_P1_CHUNK = 2000
_L = 16
_GB = 128

@functools.cache
def _mesh():
    return plsc.VectorSubcoreMesh(
        core_axis_name="c", subcore_axis_name="s",
        num_cores=_NC, num_subcores=_NS,
    )


def _wid():
    return lax.axis_index("c") * _NS + lax.axis_index("s")


def _any_lane(m):
    return plsc.all_reduce_population_count(m)[0] > 0


def _copy_body(p_ref, out_ref, cls_ref):
    x = p_ref[...]
    out_ref[...] = x
    iota = lax.broadcasted_iota(jnp.int32, x.shape, 1).astype(jnp.float32)
    cls = jnp.sum(x * iota, axis=1).astype(jnp.int32)
    cls_ref[0] = cls.reshape(cls_ref.shape[1:])


_TC_BLK = 8192


def _tc_copy(primary):
    blk = _TC_BLK
    nblk = (_N + blk - 1) // blk
    return pl.pallas_call(
        _copy_body,
        grid=(nblk,),
        in_specs=[pl.BlockSpec((blk, _CLS), lambda i: (i, 0))],
        out_specs=[
            pl.BlockSpec((blk, _CLS), lambda i: (i, 0)),
            pl.BlockSpec((1, blk // 128, 128), lambda i: (i, 0, 0)),
        ],
        out_shape=[
            jax.ShapeDtypeStruct((_N, _CLS), jnp.float32),
            jax.ShapeDtypeStruct((nblk, blk // 128, 128), jnp.int32),
        ],
    )(primary)


def _p1_body(pos_hbm, val_hbm, bits_hbm, bits_v, pos_v, val_v, sem):
    npairs = pos_hbm.shape[0]
    nchunk = npairs // _P1_CHUNK
    wid = _wid()
    base = wid * _ROWS_W

    def _zero(i, _):
        bits_v[pl.ds(i * _L, _L)] = jnp.zeros((_L,), jnp.int32)
        return 0

    lax.fori_loop(0, _ROWS_W_PAD // _L, _zero, 0)

    def _chunk(k, _):
        pltpu.async_copy(pos_hbm.at[pl.ds(k * _P1_CHUNK, _P1_CHUNK)], pos_v, sem).wait()
        pltpu.async_copy(val_hbm.at[pl.ds(k * _P1_CHUNK, _P1_CHUNK)], val_v, sem).wait()

        def _group(g, _):
            pos = pos_v[pl.ds(g * _L, _L)]
            val = val_v[pl.ds(g * _L, _L)]
            rel = pos - base
            m = (rel >= 0) & (rel < _ROWS_W)
            any_in = _any_lane(m)

            @pl.when(any_in)
            def _():
                srel = jnp.clip(rel, 0, _ROWS_W - 1)
                bitv = jnp.int32(1) << val

                def _cond(pending):
                    return _any_lane(pending)

                def _rmw(pending):
                    old = plsc.load_gather(bits_v, [srel], mask=pending)
                    plsc.store_scatter(bits_v, [srel], old | bitv, mask=pending)
                    cur = plsc.load_gather(bits_v, [srel], mask=pending)
                    return pending & ((cur & bitv) != bitv)

                lax.while_loop(_cond, _rmw, m)

            return 0

        lax.fori_loop(0, _P1_CHUNK // _L, _group, 0)
        return 0

    lax.fori_loop(0, nchunk, _chunk, 0)
    pltpu.async_copy(bits_v.at[pl.ds(0, _ROWS_W)], bits_hbm.at[wid], sem).wait()


def _sc_phase1(pos1, val1):
    return pl.kernel(
        _p1_body,
        out_type=jax.ShapeDtypeStruct((_NW, _ROWS_W), jnp.int32),
        mesh=_mesh(),
        compiler_params=pltpu.CompilerParams(needs_layout_passes=False, use_tc_tiling_on_sc=False),
        scratch_types=[
            pltpu.VMEM((_ROWS_W_PAD,), jnp.int32),
            pltpu.VMEM((_P1_CHUNK,), jnp.int32),
            pltpu.VMEM((_P1_CHUNK,), jnp.int32),
            pltpu.SemaphoreType.DMA,
        ],
    )(pos1, val1)


def _p2_body(out_ref, bits_hbm, clsp_hbm, pos2_hbm, w_hbm, b_hbm,
             p2_v, bits_b, cls_b, a1, v1, a2, v2, w_v, b_v, wb_v, t_v,
             sem1, sem2, sem3, sem4):
    nrow = p2_v.shape[0]
    wid = _wid()
    pltpu.async_copy(pos2_hbm.at[wid], p2_v, sem1).wait()
    pltpu.async_copy(w_hbm, w_v, sem1).wait()
    pltpu.async_copy(b_hbm, b_v, sem1).wait()

    lanes = lax.iota(jnp.int32, _L)

    def _wb(g, _):
        bidx = (g % 2) * _L + lanes
        wb_v[pl.ds(g * _L, _L)] = (
            w_v[pl.ds(g * _L, _L)] + plsc.load_gather(b_v, [bidx]))
        return 0

    lax.fori_loop(0, 64, _wb, 0)

    for chunk in range(2):
        cvec = chunk * _L + lanes

        def _scan(j, carry):
            best, bi = carry
            lg = plsc.load_gather(wb_v, [cvec * 32 + j])
            take = lg > best
            return jnp.where(take, lg, best), jnp.where(take, j, bi)

        best0 = plsc.load_gather(wb_v, [cvec * 32])
        _, bi = lax.fori_loop(1, _CLS, _scan, (best0, jnp.zeros((_L,), jnp.int32)))
        t_v[pl.ds(chunk * _L, _L)] = bi

    gathers = []
    for j in range(nrow):
        gathers.append(pltpu.async_copy(bits_hbm.at[p2_v.at[j]], bits_b.at[j], sem1))
        gathers.append(pltpu.async_copy(clsp_hbm.at[p2_v.at[j]], cls_b.at[j], sem2))
    for g in gathers:
        g.wait()

    def _row(j, _):
        for l in range(_GB // _L):
            sl = pl.ds(l * _L, _L)
            pos = p2_v[j, sl]
            bits = bits_b[j, sl]
            c_old = cls_b[j, sl]
            is0 = bits == 0
            one_hot = (bits & (bits - 1)) == 0
            f = bits.astype(jnp.float32)
            expo = (plsc.bitcast(f, jnp.int32) >> 23) - 127
            c_one = jnp.clip(jnp.where(is0, c_old, expo), 0, _CLS - 1)
            cls = plsc.load_gather(t_v, [c_one])

            multi = ~one_hot
            any_multi = _any_lane(multi)

            def _hard():
                def _outer(j2, carry):
                    best, bi = carry

                    def _inner(v, acc):
                        on = ((bits >> v) & 1) == 1
                        wv = plsc.load_gather(
                            w_v, [jnp.broadcast_to(v * 32 + j2, (_L,))])
                        return acc + jnp.where(on, wv, 0.0)

                    lg = lax.fori_loop(0, _CLS, _inner, jnp.zeros((_L,), jnp.float32))
                    lg = lg + plsc.load_gather(b_v, [jnp.broadcast_to(j2, (_L,))])
                    take = lg > best
                    return jnp.where(take, lg, best), jnp.where(take, j2, bi)

                init = (jnp.full((_L,), -jnp.inf, jnp.float32),
                        jnp.zeros((_L,), jnp.int32))
                _, bi = lax.fori_loop(0, _CLS, _outer, init)
                return jnp.where(multi, bi, cls)

            cls = lax.cond(any_multi, _hard, lambda: cls)

            a1[j, sl] = pos * _CLS + c_old
            v1[j, sl] = jnp.where(cls == c_old, 1.0, 0.0)
            a2[j, sl] = pos * _CLS + cls
            v2[j, sl] = jnp.ones((_L,), jnp.float32)
        return 0

    lax.fori_loop(0, nrow, _row, 0)

    scatters = []
    for j in range(nrow):
        scatters.append(pltpu.async_copy(v1.at[j], out_ref.at[a1.at[j]], sem3))
        scatters.append(pltpu.async_copy(v2.at[j], out_ref.at[a2.at[j]], sem4))
    for s in scatters:
        s.wait()


def _sc_phase2(out_flat_ref, bits, clsp, pos2, w, b_pad, nrow):
    pl.kernel(
        _p2_body,
        out_type=(),
        mesh=_mesh(),
        compiler_params=pltpu.CompilerParams(needs_layout_passes=False, use_tc_tiling_on_sc=False),
        scratch_types=[
            pltpu.VMEM((nrow, _GB), jnp.int32),
            pltpu.VMEM((nrow, _GB), jnp.int32),
            pltpu.VMEM((nrow, _GB), jnp.int32),
            pltpu.VMEM((nrow, _GB), jnp.int32),
            pltpu.VMEM((nrow, _GB), jnp.float32),
            pltpu.VMEM((nrow, _GB), jnp.int32),
            pltpu.VMEM((nrow, _GB), jnp.float32),
            pltpu.VMEM((1024,), jnp.float32),
            pltpu.VMEM((32,), jnp.float32),
            pltpu.VMEM((1024,), jnp.float32),
            pltpu.VMEM((32,), jnp.int32),
            pltpu.SemaphoreType.DMA,
            pltpu.SemaphoreType.DMA,
            pltpu.SemaphoreType.DMA,
            pltpu.SemaphoreType.DMA,
        ],
    )(out_flat_ref, bits, clsp, pos2, w, b_pad)


def kernel(primary, W, b, positions1, values1, positions2):
    p1 = positions1.shape[0]
    p2 = positions2.shape[0]

    pad1 = (-p1) % _P1_CHUNK
    if pad1:
        positions1 = jnp.concatenate(
            [positions1, jnp.full((pad1,), -1, positions1.dtype)])
        values1 = jnp.concatenate([values1, jnp.zeros((pad1,), values1.dtype)])

    pad2 = (-p2) % (_NW * _GB)
    if pad2:
        positions2 = jnp.concatenate(
            [positions2, jnp.broadcast_to(positions2[:1], (pad2,))])
    nrow = positions2.shape[0] // (_NW * _GB)
    pos2 = positions2.reshape(_NW, nrow, _GB).astype(jnp.int32)

    b_pad = jnp.concatenate([b, jnp.zeros((32 - _CLS,), b.dtype)])
    w_pad = jnp.zeros((32, 32), W.dtype).at[:_CLS, :_CLS].set(W).reshape(1024)

    out0, cls3 = _tc_copy(primary)
    clsp = cls3.reshape(-1)[:_N]
    bits = _sc_phase1(positions1.astype(jnp.int32),
                      values1.astype(jnp.int32)).reshape(_N)

    oref = jax.new_ref(out0.reshape(_N * _CLS))
    _sc_phase2(oref, bits, clsp, pos2, w_pad, b_pad, nrow)
    return oref[...].reshape(_N, _CLS)


# speedup vs baseline: 8.9704x; 1.2694x over previous
import functools

import jax
import jax.numpy as jnp
from jax import lax
from jax.experimental import pallas as pl
from jax.experimental.pallas import tpu as pltpu
from jax.experimental.pallas import tpu_sc as plsc

_N = 1000000
_CLS = 20
_NC = 2
_NS = 16
_NW = _NC * _NS
_ROWS_W = _N // _NW
_ROWS_W_PAD = ((_ROWS_W + 15) // 16) * 16
_RCHUNK = 1250
_P1_CHUNK = 2000
_L = 16
_GB = 128

_SC_PARAMS = pltpu.CompilerParams(
    needs_layout_passes=False, use_tc_tiling_on_sc=False)


@functools.cache
def _mesh():
    return plsc.VectorSubcoreMesh(
        core_axis_name="c", subcore_axis_name="s",
        num_cores=_NC, num_subcores=_NS,
    )


def _wid():
    return lax.axis_index("c") * _NS + lax.axis_index("s")


def _any_lane(m):
    return plsc.all_reduce_population_count(m)[0] > 0


_TC_BLK = 160 * 1024


def _copy_body(p_ref, out_ref):
    out_ref[...] = p_ref[...]


def _tc_copy(primary_flat):
    total = primary_flat.shape[0]
    nblk = (total + _TC_BLK - 1) // _TC_BLK
    return pl.pallas_call(
        _copy_body,
        grid=(nblk,),
        in_specs=[pl.BlockSpec((_TC_BLK,), lambda i: (i,))],
        out_specs=pl.BlockSpec((_TC_BLK,), lambda i: (i,)),
        out_shape=jax.ShapeDtypeStruct((total,), jnp.float32),
    )(primary_flat)


def _p1_body(prim_hbm, pos_hbm, val_hbm, comb_hbm,
             comb_v, prim_c, pos_v, val_v, sem):
    npairs = pos_hbm.shape[0]
    nchunk = npairs // _P1_CHUNK
    wid = _wid()
    base = wid * _ROWS_W
    lanes = lax.iota(jnp.int32, _L)
    cwords = _RCHUNK * _CLS

    ngrp = (_RCHUNK + _L - 1) // _L

    def _rchunk(c, _):
        pltpu.async_copy(
            prim_hbm.at[pl.ds(base * _CLS + c * cwords, cwords)], prim_c, sem
        ).wait()

        def _grp(g, _):
            rowv = g * _L + lanes

            def _dot(k, acc):
                idx = jnp.minimum(rowv * _CLS + k, cwords - 1)
                return acc + plsc.load_gather(prim_c, [idx]) * k.astype(jnp.float32)

            cls = lax.fori_loop(
                1, _CLS, _dot, jnp.zeros((_L,), jnp.float32)).astype(jnp.int32)
            comb_v[pl.ds(c * _RCHUNK + g * _L, _L)] = cls
            return 0

        lax.fori_loop(0, ngrp, _grp, 0)
        return 0

    lax.fori_loop(0, _ROWS_W // _RCHUNK, _rchunk, 0)

    def _chunk(k, _):
        pltpu.async_copy(pos_hbm.at[pl.ds(k * _P1_CHUNK, _P1_CHUNK)], pos_v, sem).wait()
        pltpu.async_copy(val_hbm.at[pl.ds(k * _P1_CHUNK, _P1_CHUNK)], val_v, sem).wait()

        def _group(g, _):
            pos = pos_v[pl.ds(g * _L, _L)]
            val = val_v[pl.ds(g * _L, _L)]
            rel = pos - base
            m = (rel >= 0) & (rel < _ROWS_W)
            any_in = _any_lane(m)

            @pl.when(any_in)
            def _():
                srel = jnp.clip(rel, 0, _ROWS_W - 1)
                bitv = jnp.int32(1) << (val + 5)

                def _cond(pending):
                    return _any_lane(pending)

                def _rmw(pending):
                    old = plsc.load_gather(comb_v, [srel], mask=pending)
                    plsc.store_scatter(comb_v, [srel], old | bitv, mask=pending)
                    cur = plsc.load_gather(comb_v, [srel], mask=pending)
                    return pending & ((cur & bitv) != bitv)

                lax.while_loop(_cond, _rmw, m)

            return 0

        lax.fori_loop(0, _P1_CHUNK // _L, _group, 0)
        return 0

    lax.fori_loop(0, nchunk, _chunk, 0)
    pltpu.async_copy(comb_v.at[pl.ds(0, _ROWS_W)], comb_hbm.at[wid], sem).wait()


def _sc_phase1(prim_flat, pos1, val1):
    return pl.kernel(
        _p1_body,
        out_type=jax.ShapeDtypeStruct((_NW, _ROWS_W), jnp.int32),
        mesh=_mesh(),
        compiler_params=_SC_PARAMS,
        scratch_types=[
            pltpu.VMEM((_ROWS_W_PAD,), jnp.int32),
            pltpu.VMEM((_RCHUNK * _CLS,), jnp.float32),
            pltpu.VMEM((_P1_CHUNK,), jnp.int32),
            pltpu.VMEM((_P1_CHUNK,), jnp.int32),
            pltpu.SemaphoreType.DMA,
        ],
    )(prim_flat, pos1, val1)


def _p2_body(out_ref, comb_hbm, pos2_hbm, w_hbm, b_hbm,
             p2_v, comb_b, a1, v1, a2, v2, w_v, b_v, wb_v, t_v,
             sem1, sem3, sem4):
    nrow = p2_v.shape[0]
    wid = _wid()
    pltpu.async_copy(pos2_hbm.at[wid], p2_v, sem1).wait()
    pltpu.async_copy(w_hbm, w_v, sem1).wait()
    pltpu.async_copy(b_hbm, b_v, sem1).wait()

    lanes = lax.iota(jnp.int32, _L)

    def _wb(g, _):
        bidx = (g % 2) * _L + lanes
        wb_v[pl.ds(g * _L, _L)] = (
            w_v[pl.ds(g * _L, _L)] + plsc.load_gather(b_v, [bidx]))
        return 0

    lax.fori_loop(0, 64, _wb, 0)

    for chunk in range(2):
        cvec = chunk * _L + lanes

        def _scan(j, carry):
            best, bi = carry
            lg = plsc.load_gather(wb_v, [cvec * 32 + j])
            take = lg > best
            return jnp.where(take, lg, best), jnp.where(take, j, bi)

        best0 = plsc.load_gather(wb_v, [cvec * 32])
        _, bi = lax.fori_loop(1, _CLS, _scan, (best0, jnp.zeros((_L,), jnp.int32)))
        t_v[pl.ds(chunk * _L, _L)] = bi

    gathers = []
    for j in range(nrow):
        gathers.append(pltpu.async_copy(comb_hbm.at[p2_v.at[j]], comb_b.at[j], sem1))
    for g in gathers:
        g.wait()

    def _row(j, _):
        for l in range(_GB // _L):
            sl = pl.ds(l * _L, _L)
            pos = p2_v[j, sl]
            comb = comb_b[j, sl]
            c_old = comb & 31
            bits = comb >> 5
            is0 = bits == 0
            one_hot = (bits & (bits - 1)) == 0
            f = bits.astype(jnp.float32)
            expo = (plsc.bitcast(f, jnp.int32) >> 23) - 127
            c_one = jnp.clip(jnp.where(is0, c_old, expo), 0, _CLS - 1)
            cls = plsc.load_gather(t_v, [c_one])

            multi = ~one_hot
            any_multi = _any_lane(multi)

            def _hard():
                def _outer(j2, carry):
                    best, bi = carry

                    def _inner(v, acc):
                        on = ((bits >> v) & 1) == 1
                        wv = plsc.load_gather(
                            w_v, [jnp.broadcast_to(v * 32 + j2, (_L,))])
                        return acc + jnp.where(on, wv, 0.0)

                    lg = lax.fori_loop(0, _CLS, _inner, jnp.zeros((_L,), jnp.float32))
                    lg = lg + plsc.load_gather(b_v, [jnp.broadcast_to(j2, (_L,))])
                    take = lg > best
                    return jnp.where(take, lg, best), jnp.where(take, j2, bi)

                init = (jnp.full((_L,), -jnp.inf, jnp.float32),
                        jnp.zeros((_L,), jnp.int32))
                _, bi = lax.fori_loop(0, _CLS, _outer, init)
                return jnp.where(multi, bi, cls)

            cls = lax.cond(any_multi, _hard, lambda: cls)

            a1[j, sl] = pos * _CLS + c_old
            v1[j, sl] = jnp.where(cls == c_old, jnp.float32(1.0), jnp.float32(0.0))
            a2[j, sl] = pos * _CLS + cls
            v2[j, sl] = jnp.ones((_L,), jnp.float32)
        return 0

    lax.fori_loop(0, nrow, _row, 0)

    scatters = []
    for j in range(nrow):
        scatters.append(pltpu.async_copy(v1.at[j], out_ref.at[a1.at[j]], sem3))
        scatters.append(pltpu.async_copy(v2.at[j], out_ref.at[a2.at[j]], sem4))
    for s in scatters:
        s.wait()


def _sc_phase2(out_flat_ref, comb, pos2, w, b_pad, nrow):
    pl.kernel(
        _p2_body,
        out_type=(),
        mesh=_mesh(),
        compiler_params=_SC_PARAMS,
        scratch_types=[
            pltpu.VMEM((nrow, _GB), jnp.int32),
            pltpu.VMEM((nrow, _GB), jnp.int32),
            pltpu.VMEM((nrow, _GB), jnp.int32),
            pltpu.VMEM((nrow, _GB), jnp.float32),
            pltpu.VMEM((nrow, _GB), jnp.int32),
            pltpu.VMEM((nrow, _GB), jnp.float32),
            pltpu.VMEM((1024,), jnp.float32),
            pltpu.VMEM((32,), jnp.float32),
            pltpu.VMEM((1024,), jnp.float32),
            pltpu.VMEM((32,), jnp.int32),
            pltpu.SemaphoreType.DMA,
            pltpu.SemaphoreType.DMA,
            pltpu.SemaphoreType.DMA,
        ],
    )(out_flat_ref, comb, pos2, w, b_pad)


def kernel(primary, W, b, positions1, values1, positions2):
    p1 = positions1.shape[0]
    p2 = positions2.shape[0]

    pad1 = (-p1) % _P1_CHUNK
    if pad1:
        positions1 = jnp.concatenate(
            [positions1, jnp.full((pad1,), -1, positions1.dtype)])
        values1 = jnp.concatenate([values1, jnp.zeros((pad1,), values1.dtype)])

    pad2 = (-p2) % (_NW * _GB)
    if pad2:
        reps = -(-pad2 // p2)
        extra = jnp.tile(positions2, reps)[:pad2]
        positions2 = jnp.concatenate([positions2, extra])
    nrow = positions2.shape[0] // (_NW * _GB)
    pos2 = positions2.reshape(_NW, nrow, _GB).astype(jnp.int32)

    b_pad = jnp.concatenate([b, jnp.zeros((32 - _CLS,), b.dtype)])
    w_pad = jnp.zeros((32, 32), W.dtype).at[:_CLS, :_CLS].set(W).reshape(1024)

    prim_flat = primary.reshape(_N * _CLS)
    out0 = _tc_copy(prim_flat)
    comb = _sc_phase1(prim_flat, positions1.astype(jnp.int32),
                      values1.astype(jnp.int32)).reshape(_N)

    oref = jax.new_ref(out0)
    _sc_phase2(oref, comb, pos2, w_pad, b_pad, nrow)
    return oref[...].reshape(_N, _CLS)


# speedup vs baseline: 56.5505x; 6.3041x over previous
import functools

import jax
import jax.numpy as jnp
from jax import lax
from jax.experimental import pallas as pl
from jax.experimental.pallas import tpu as pltpu
from jax.experimental.pallas import tpu_sc as plsc

_N = 1000000
_CLS = 20
_NC = 2
_NS = 16
_NW = _NC * _NS
_ROWS_W = _N // _NW
_ROWS_W_PAD = ((_ROWS_W + 15) // 16) * 16
_P1_CHUNK = 2000
_L = 16
_GB = 128
_TCB = 32768

_SC_PARAMS = pltpu.CompilerParams(
    needs_layout_passes=False, use_tc_tiling_on_sc=False)


@functools.cache
def _mesh():
    return plsc.VectorSubcoreMesh(
        core_axis_name="c", subcore_axis_name="s",
        num_cores=_NC, num_subcores=_NS,
    )


def _wid():
    return lax.axis_index("c") * _NS + lax.axis_index("s")


def _any_lane(m):
    return plsc.all_reduce_population_count(m)[0] > 0


def _prep_body(p_ref, cls_ref, upd_ref):
    x = p_ref[...]
    ci = lax.broadcasted_iota(jnp.int32, x.shape, 0).astype(jnp.float32)
    cls_ref[...] = jnp.sum(x * ci, axis=0)
    upd_ref[...] = jnp.full(upd_ref.shape, -1.0, jnp.float32)


def _tc_prep(primT):
    nblk = (_N + _TCB - 1) // _TCB
    return pl.pallas_call(
        _prep_body,
        grid=(nblk,),
        in_specs=[pl.BlockSpec((_CLS, _TCB), lambda i: (0, i))],
        out_specs=[
            pl.BlockSpec((_TCB,), lambda i: (i,)),
            pl.BlockSpec((_TCB,), lambda i: (i,)),
        ],
        out_shape=[
            jax.ShapeDtypeStruct((_N,), jnp.float32),
            jax.ShapeDtypeStruct((_N,), jnp.float32),
        ],
    )(primT)


def _apply_body(p_ref, u_ref, out_ref):
    x = p_ref[...]
    u = u_ref[...]
    ub = jnp.broadcast_to(u[None, :], x.shape)
    ci = lax.broadcasted_iota(jnp.int32, x.shape, 0).astype(jnp.float32)
    out_ref[...] = jnp.where(ub < 0.0, x, (ci == ub).astype(jnp.float32))


def _tc_apply(primT, upd):
    nblk = (_N + _TCB - 1) // _TCB
    return pl.pallas_call(
        _apply_body,
        grid=(nblk,),
        in_specs=[
            pl.BlockSpec((_CLS, _TCB), lambda i: (0, i)),
            pl.BlockSpec((_TCB,), lambda i: (i,)),
        ],
        out_specs=pl.BlockSpec((_CLS, _TCB), lambda i: (0, i)),
        out_shape=jax.ShapeDtypeStruct((_CLS, _N), jnp.float32),
    )(primT, upd)


def _p1_body(pos_hbm, val_hbm, bits_hbm, bits_v, pos_v, val_v, sem):
    npairs = pos_hbm.shape[0]
    nchunk = npairs // _P1_CHUNK
    wid = _wid()
    base = wid * _ROWS_W

    def _zero(i, _):
        bits_v[pl.ds(i * _L, _L)] = jnp.zeros((_L,), jnp.int32)
        return 0

    lax.fori_loop(0, _ROWS_W_PAD // _L, _zero, 0)

    def _chunk(k, _):
        pltpu.async_copy(pos_hbm.at[pl.ds(k * _P1_CHUNK, _P1_CHUNK)], pos_v, sem).wait()
        pltpu.async_copy(val_hbm.at[pl.ds(k * _P1_CHUNK, _P1_CHUNK)], val_v, sem).wait()

        def _group(g, _):
            pos = pos_v[pl.ds(g * _L, _L)]
            val = val_v[pl.ds(g * _L, _L)]
            rel = pos - base
            m = (rel >= 0) & (rel < _ROWS_W)
            any_in = _any_lane(m)

            @pl.when(any_in)
            def _():
                srel = jnp.clip(rel, 0, _ROWS_W - 1)
                bitv = jnp.int32(1) << val

                def _cond(pending):
                    return _any_lane(pending)

                def _rmw(pending):
                    old = plsc.load_gather(bits_v, [srel], mask=pending)
                    plsc.store_scatter(bits_v, [srel], old | bitv, mask=pending)
                    cur = plsc.load_gather(bits_v, [srel], mask=pending)
                    return pending & ((cur & bitv) != bitv)

                lax.while_loop(_cond, _rmw, m)

            return 0

        lax.fori_loop(0, _P1_CHUNK // _L, _group, 0)
        return 0

    lax.fori_loop(0, nchunk, _chunk, 0)
    pltpu.async_copy(bits_v.at[pl.ds(0, _ROWS_W)], bits_hbm.at[wid], sem).wait()


def _sc_phase1(pos1, val1):
    return pl.kernel(
        _p1_body,
        out_type=jax.ShapeDtypeStruct((_NW, _ROWS_W), jnp.int32),
        mesh=_mesh(),
        compiler_params=_SC_PARAMS,
        scratch_types=[
            pltpu.VMEM((_ROWS_W_PAD,), jnp.int32),
            pltpu.VMEM((_P1_CHUNK,), jnp.int32),
            pltpu.VMEM((_P1_CHUNK,), jnp.int32),
            pltpu.SemaphoreType.DMA,
        ],
    )(pos1, val1)


def _p2_body(upd_ref, bits_hbm, clsp_hbm, pos2_hbm, w_hbm, b_hbm,
             p2_v, bits_b, cls_b, av, vv, w_v, b_v, wb_v, t_v,
             sem1, sem2, sem3):
    nrow = p2_v.shape[0]
    wid = _wid()
    pltpu.async_copy(pos2_hbm.at[wid], p2_v, sem1).wait()
    pltpu.async_copy(w_hbm, w_v, sem1).wait()
    pltpu.async_copy(b_hbm, b_v, sem1).wait()

    lanes = lax.iota(jnp.int32, _L)

    def _wb(g, _):
        bidx = (g % 2) * _L + lanes
        wb_v[pl.ds(g * _L, _L)] = (
            w_v[pl.ds(g * _L, _L)] + plsc.load_gather(b_v, [bidx]))
        return 0

    lax.fori_loop(0, 64, _wb, 0)

    for chunk in range(2):
        cvec = chunk * _L + lanes

        def _scan(j, carry):
            best, bi = carry
            lg = plsc.load_gather(wb_v, [cvec * 32 + j])
            take = lg > best
            return jnp.where(take, lg, best), jnp.where(take, j, bi)

        best0 = plsc.load_gather(wb_v, [cvec * 32])
        _, bi = lax.fori_loop(1, _CLS, _scan, (best0, jnp.zeros((_L,), jnp.int32)))
        t_v[pl.ds(chunk * _L, _L)] = bi

    gathers = []
    for j in range(nrow):
        gathers.append(pltpu.async_copy(bits_hbm.at[p2_v.at[j]], bits_b.at[j], sem1))
        gathers.append(pltpu.async_copy(clsp_hbm.at[p2_v.at[j]], cls_b.at[j], sem2))
    for g in gathers:
        g.wait()

    def _row(j, _):
        for l in range(_GB // _L):
            sl = pl.ds(l * _L, _L)
            pos = p2_v[j, sl]
            bits = bits_b[j, sl]
            c_old = cls_b[j, sl].astype(jnp.int32)
            is0 = bits == 0
            one_hot = (bits & (bits - 1)) == 0
            f = bits.astype(jnp.float32)
            expo = (plsc.bitcast(f, jnp.int32) >> 23) - 127
            c_one = jnp.clip(jnp.where(is0, c_old, expo), 0, _CLS - 1)
            cls = plsc.load_gather(t_v, [c_one])

            multi = ~one_hot
            any_multi = _any_lane(multi)

            def _hard():
                def _outer(j2, carry):
                    best, bi = carry

                    def _inner(v, acc):
                        on = ((bits >> v) & 1) == 1
                        wv = plsc.load_gather(
                            w_v, [jnp.broadcast_to(v * 32 + j2, (_L,))])
                        return acc + jnp.where(on, wv, 0.0)

                    lg = lax.fori_loop(0, _CLS, _inner, jnp.zeros((_L,), jnp.float32))
                    lg = lg + plsc.load_gather(b_v, [jnp.broadcast_to(j2, (_L,))])
                    take = lg > best
                    return jnp.where(take, lg, best), jnp.where(take, j2, bi)

                init = (jnp.full((_L,), -jnp.inf, jnp.float32),
                        jnp.zeros((_L,), jnp.int32))
                _, bi = lax.fori_loop(0, _CLS, _outer, init)
                return jnp.where(multi, bi, cls)

            cls = lax.cond(any_multi, _hard, lambda: cls)

            av[j, sl] = pos
            vv[j, sl] = cls.astype(jnp.float32)
        return 0

    lax.fori_loop(0, nrow, _row, 0)

    scatters = []
    for j in range(nrow):
        scatters.append(pltpu.async_copy(vv.at[j], upd_ref.at[av.at[j]], sem3))
    for s in scatters:
        s.wait()


def _sc_phase2(upd_flat_ref, bits, clsp, pos2, w, b_pad, nrow):
    pl.kernel(
        _p2_body,
        out_type=(),
        mesh=_mesh(),
        compiler_params=_SC_PARAMS,
        scratch_types=[
            pltpu.VMEM((nrow, _GB), jnp.int32),
            pltpu.VMEM((nrow, _GB), jnp.int32),
            pltpu.VMEM((nrow, _GB), jnp.float32),
            pltpu.VMEM((nrow, _GB), jnp.int32),
            pltpu.VMEM((nrow, _GB), jnp.float32),
            pltpu.VMEM((1024,), jnp.float32),
            pltpu.VMEM((32,), jnp.float32),
            pltpu.VMEM((1024,), jnp.float32),
            pltpu.VMEM((32,), jnp.int32),
            pltpu.SemaphoreType.DMA,
            pltpu.SemaphoreType.DMA,
            pltpu.SemaphoreType.DMA,
        ],
    )(upd_flat_ref, bits, clsp, pos2, w, b_pad)


def kernel(primary, W, b, positions1, values1, positions2):
    p1 = positions1.shape[0]
    p2 = positions2.shape[0]

    pad1 = (-p1) % _P1_CHUNK
    if pad1:
        positions1 = jnp.concatenate(
            [positions1, jnp.full((pad1,), -1, positions1.dtype)])
        values1 = jnp.concatenate([values1, jnp.zeros((pad1,), values1.dtype)])

    pad2 = (-p2) % (_NW * _GB)
    if pad2:
        reps = -(-pad2 // p2)
        extra = jnp.tile(positions2, reps)[:pad2]
        positions2 = jnp.concatenate([positions2, extra])
    nrow = positions2.shape[0] // (_NW * _GB)
    pos2 = positions2.reshape(_NW, nrow, _GB).astype(jnp.int32)

    b_pad = jnp.concatenate([b, jnp.zeros((32 - _CLS,), b.dtype)])
    w_pad = jnp.zeros((32, 32), W.dtype).at[:_CLS, :_CLS].set(W).reshape(1024)

    primT = primary.T
    clsp, upd0 = _tc_prep(primT)
    bits = _sc_phase1(positions1.astype(jnp.int32),
                      values1.astype(jnp.int32)).reshape(_N)

    uref = jax.new_ref(upd0)
    _sc_phase2(uref, bits, clsp, pos2, w_pad, b_pad, nrow)
    outT = _tc_apply(primT, uref[...])
    return outT.T


# speedup vs baseline: 63.5400x; 1.1236x over previous
import functools

import jax
import jax.numpy as jnp
from jax import lax
from jax.experimental import pallas as pl
from jax.experimental.pallas import tpu as pltpu
from jax.experimental.pallas import tpu_sc as plsc

_N = 1000000
_CLS = 20
_NC = 2
_NS = 16
_NW = _NC * _NS
_ROWS_W = _N // _NW
_ROWS_W_PAD = ((_ROWS_W + 15) // 16) * 16
_P1_CHUNK = 2000
_L = 16
_GB = 128
_TCB = 32768

_SC_PARAMS = pltpu.CompilerParams(
    needs_layout_passes=False, use_tc_tiling_on_sc=False)


@functools.cache
def _mesh():
    return plsc.VectorSubcoreMesh(
        core_axis_name="c", subcore_axis_name="s",
        num_cores=_NC, num_subcores=_NS,
    )


def _wid():
    return lax.axis_index("c") * _NS + lax.axis_index("s")


def _any_lane(m):
    return plsc.all_reduce_population_count(m)[0] > 0


def _prep_body(p_ref, cls_ref, upd_ref):
    x = p_ref[...]
    ci = lax.broadcasted_iota(jnp.int32, x.shape, 0).astype(jnp.float32)
    cls_ref[...] = jnp.sum(x * ci, axis=0)
    upd_ref[...] = jnp.full(upd_ref.shape, -1.0, jnp.float32)


def _tc_prep(primT):
    nblk = (_N + _TCB - 1) // _TCB
    return pl.pallas_call(
        _prep_body,
        grid=(nblk,),
        in_specs=[pl.BlockSpec((_CLS, _TCB), lambda i: (0, i))],
        out_specs=[
            pl.BlockSpec((_TCB,), lambda i: (i,)),
            pl.BlockSpec((_TCB,), lambda i: (i,)),
        ],
        out_shape=[
            jax.ShapeDtypeStruct((_N,), jnp.float32),
            jax.ShapeDtypeStruct((_N,), jnp.float32),
        ],
    )(primT)


def _apply_body(p_ref, u_ref, out_ref):
    x = p_ref[...]
    u = u_ref[...]
    ub = jnp.broadcast_to(u[None, :], x.shape)
    ci = lax.broadcasted_iota(jnp.int32, x.shape, 0).astype(jnp.float32)
    out_ref[...] = jnp.where(ub < 0.0, x, (ci == ub).astype(jnp.float32))


def _tc_apply(primT, upd):
    nblk = (_N + _TCB - 1) // _TCB
    return pl.pallas_call(
        _apply_body,
        grid=(nblk,),
        in_specs=[
            pl.BlockSpec((_CLS, _TCB), lambda i: (0, i)),
            pl.BlockSpec((_TCB,), lambda i: (i,)),
        ],
        out_specs=pl.BlockSpec((_CLS, _TCB), lambda i: (0, i)),
        out_shape=jax.ShapeDtypeStruct((_CLS, _N), jnp.float32),
    )(primT, upd)


def _p1_body(pos_hbm, val_hbm, bits_hbm, bits_v, pos_v, val_v, sem):
    npairs = pos_hbm.shape[0]
    nchunk = npairs // _P1_CHUNK
    wid = _wid()
    base = wid * _ROWS_W

    def _zero(i, _):
        bits_v[pl.ds(i * _L, _L)] = jnp.zeros((_L,), jnp.int32)
        return 0

    lax.fori_loop(0, _ROWS_W_PAD // _L, _zero, 0)

    def _start(k):
        par = (k % 2) * _P1_CHUNK
        sl = pl.ds(k * _P1_CHUNK, _P1_CHUNK)
        dsl = pl.ds(par, _P1_CHUNK)
        pltpu.make_async_copy(pos_hbm.at[sl], pos_v.at[dsl], sem).start()
        pltpu.make_async_copy(val_hbm.at[sl], val_v.at[dsl], sem).start()

    def _wait(k):
        par = (k % 2) * _P1_CHUNK
        sl = pl.ds(k * _P1_CHUNK, _P1_CHUNK)
        dsl = pl.ds(par, _P1_CHUNK)
        pltpu.make_async_copy(pos_hbm.at[sl], pos_v.at[dsl], sem).wait()
        pltpu.make_async_copy(val_hbm.at[sl], val_v.at[dsl], sem).wait()

    _start(0)

    def _chunk(k, _):
        @pl.when(k + 1 < nchunk)
        def _():
            _start(k + 1)

        _wait(k)
        par = (k % 2) * _P1_CHUNK

        def _group(g, _):
            off = par + g * 2 * _L
            pos_a = pos_v[pl.ds(off, _L)]
            pos_b = pos_v[pl.ds(off + _L, _L)]
            rel_a = pos_a - base
            rel_b = pos_b - base
            m_a = (rel_a >= 0) & (rel_a < _ROWS_W)
            m_b = (rel_b >= 0) & (rel_b < _ROWS_W)
            any_in = _any_lane(m_a | m_b)

            @pl.when(any_in)
            def _():
                for (rel, m, voff) in ((rel_a, m_a, off), (rel_b, m_b, off + _L)):
                    val = val_v[pl.ds(voff, _L)]
                    srel = jnp.clip(rel, 0, _ROWS_W - 1)
                    bitv = jnp.int32(1) << val

                    def _cond(pending):
                        return _any_lane(pending)

                    def _rmw(pending):
                        old = plsc.load_gather(bits_v, [srel], mask=pending)
                        plsc.store_scatter(bits_v, [srel], old | bitv, mask=pending)
                        cur = plsc.load_gather(bits_v, [srel], mask=pending)
                        return pending & ((cur & bitv) != bitv)

                    lax.while_loop(_cond, _rmw, m)

            return 0

        lax.fori_loop(0, _P1_CHUNK // (2 * _L), _group, 0)
        return 0

    lax.fori_loop(0, nchunk, _chunk, 0)
    pltpu.async_copy(bits_v.at[pl.ds(0, _ROWS_W)], bits_hbm.at[wid], sem).wait()


def _sc_phase1(pos1, val1):
    return pl.kernel(
        _p1_body,
        out_type=jax.ShapeDtypeStruct((_NW, _ROWS_W), jnp.int32),
        mesh=_mesh(),
        compiler_params=_SC_PARAMS,
        scratch_types=[
            pltpu.VMEM((_ROWS_W_PAD,), jnp.int32),
            pltpu.VMEM((2 * _P1_CHUNK,), jnp.int32),
            pltpu.VMEM((2 * _P1_CHUNK,), jnp.int32),
            pltpu.SemaphoreType.DMA,
        ],
    )(pos1, val1)


def _p2_body(upd_ref, bits_hbm, clsp_hbm, pos2_hbm, w_hbm, b_hbm,
             p2_v, bits_b, cls_b, av, vv, w_v, b_v, wb_v, t_v,
             sem1, sem2, sem3):
    nrow = p2_v.shape[0]
    wid = _wid()
    pltpu.async_copy(pos2_hbm.at[wid], p2_v, sem1).wait()
    pltpu.async_copy(w_hbm, w_v, sem1).wait()
    pltpu.async_copy(b_hbm, b_v, sem1).wait()

    lanes = lax.iota(jnp.int32, _L)

    def _wb(g, _):
        bidx = (g % 2) * _L + lanes
        wb_v[pl.ds(g * _L, _L)] = (
            w_v[pl.ds(g * _L, _L)] + plsc.load_gather(b_v, [bidx]))
        return 0

    lax.fori_loop(0, 64, _wb, 0)

    for chunk in range(2):
        cvec = chunk * _L + lanes

        def _scan(j, carry):
            best, bi = carry
            lg = plsc.load_gather(wb_v, [cvec * 32 + j])
            take = lg > best
            return jnp.where(take, lg, best), jnp.where(take, j, bi)

        best0 = plsc.load_gather(wb_v, [cvec * 32])
        _, bi = lax.fori_loop(1, _CLS, _scan, (best0, jnp.zeros((_L,), jnp.int32)))
        t_v[pl.ds(chunk * _L, _L)] = bi

    gathers = []
    for j in range(nrow):
        gathers.append(pltpu.async_copy(bits_hbm.at[p2_v.at[j]], bits_b.at[j], sem1))
        gathers.append(pltpu.async_copy(clsp_hbm.at[p2_v.at[j]], cls_b.at[j], sem2))
    for g in gathers:
        g.wait()

    def _row(j, _):
        for l in range(_GB // _L):
            sl = pl.ds(l * _L, _L)
            pos = p2_v[j, sl]
            bits = bits_b[j, sl]
            c_old = cls_b[j, sl].astype(jnp.int32)
            is0 = bits == 0
            one_hot = (bits & (bits - 1)) == 0
            f = bits.astype(jnp.float32)
            expo = (plsc.bitcast(f, jnp.int32) >> 23) - 127
            c_one = jnp.clip(jnp.where(is0, c_old, expo), 0, _CLS - 1)
            cls = plsc.load_gather(t_v, [c_one])

            multi = ~one_hot
            any_multi = _any_lane(multi)

            def _hard():
                def _outer(j2, carry):
                    best, bi = carry

                    def _inner(v, acc):
                        on = ((bits >> v) & 1) == 1
                        wv = plsc.load_gather(
                            w_v, [jnp.broadcast_to(v * 32 + j2, (_L,))])
                        return acc + jnp.where(on, wv, 0.0)

                    lg = lax.fori_loop(0, _CLS, _inner, jnp.zeros((_L,), jnp.float32))
                    lg = lg + plsc.load_gather(b_v, [jnp.broadcast_to(j2, (_L,))])
                    take = lg > best
                    return jnp.where(take, lg, best), jnp.where(take, j2, bi)

                init = (jnp.full((_L,), -jnp.inf, jnp.float32),
                        jnp.zeros((_L,), jnp.int32))
                _, bi = lax.fori_loop(0, _CLS, _outer, init)
                return jnp.where(multi, bi, cls)

            cls = lax.cond(any_multi, _hard, lambda: cls)

            av[j, sl] = pos
            vv[j, sl] = cls.astype(jnp.float32)
        return 0

    lax.fori_loop(0, nrow, _row, 0)

    scatters = []
    for j in range(nrow):
        scatters.append(pltpu.async_copy(vv.at[j], upd_ref.at[av.at[j]], sem3))
    for s in scatters:
        s.wait()


def _sc_phase2(upd_flat_ref, bits, clsp, pos2, w, b_pad, nrow):
    pl.kernel(
        _p2_body,
        out_type=(),
        mesh=_mesh(),
        compiler_params=_SC_PARAMS,
        scratch_types=[
            pltpu.VMEM((nrow, _GB), jnp.int32),
            pltpu.VMEM((nrow, _GB), jnp.int32),
            pltpu.VMEM((nrow, _GB), jnp.float32),
            pltpu.VMEM((nrow, _GB), jnp.int32),
            pltpu.VMEM((nrow, _GB), jnp.float32),
            pltpu.VMEM((1024,), jnp.float32),
            pltpu.VMEM((32,), jnp.float32),
            pltpu.VMEM((1024,), jnp.float32),
            pltpu.VMEM((32,), jnp.int32),
            pltpu.SemaphoreType.DMA,
            pltpu.SemaphoreType.DMA,
            pltpu.SemaphoreType.DMA,
        ],
    )(upd_flat_ref, bits, clsp, pos2, w, b_pad)


def kernel(primary, W, b, positions1, values1, positions2):
    p1 = positions1.shape[0]
    p2 = positions2.shape[0]

    pad1 = (-p1) % _P1_CHUNK
    if pad1:
        positions1 = jnp.concatenate(
            [positions1, jnp.full((pad1,), -1, positions1.dtype)])
        values1 = jnp.concatenate([values1, jnp.zeros((pad1,), values1.dtype)])

    pad2 = (-p2) % (_NW * _GB)
    if pad2:
        reps = -(-pad2 // p2)
        extra = jnp.tile(positions2, reps)[:pad2]
        positions2 = jnp.concatenate([positions2, extra])
    nrow = positions2.shape[0] // (_NW * _GB)
    pos2 = positions2.reshape(_NW, nrow, _GB).astype(jnp.int32)

    b_pad = jnp.concatenate([b, jnp.zeros((32 - _CLS,), b.dtype)])
    w_pad = jnp.zeros((32, 32), W.dtype).at[:_CLS, :_CLS].set(W).reshape(1024)

    primT = primary.T
    clsp, upd0 = _tc_prep(primT)
    bits = _sc_phase1(positions1.astype(jnp.int32),
                      values1.astype(jnp.int32)).reshape(_N)

    uref = jax.new_ref(upd0)
    _sc_phase2(uref, bits, clsp, pos2, w_pad, b_pad, nrow)
    outT = _tc_apply(primT, uref[...])
    return outT.T


# speedup vs baseline: 63.7938x; 1.0040x over previous
import functools

import jax
import jax.numpy as jnp
from jax import lax
from jax.experimental import pallas as pl
from jax.experimental.pallas import tpu as pltpu
from jax.experimental.pallas import tpu_sc as plsc

_N = 1000000
_CLS = 20
_NC = 2
_NS = 16
_NW = _NC * _NS
_ROWS_W = _N // _NW
_ROWS_W_PAD = ((_ROWS_W + 15) // 16) * 16
_P1_CHUNK = 2048
_L = 16
_GB = 128
_TCB = 32768

_SC_PARAMS = pltpu.CompilerParams(
    needs_layout_passes=False, use_tc_tiling_on_sc=False)


@functools.cache
def _mesh():
    return plsc.VectorSubcoreMesh(
        core_axis_name="c", subcore_axis_name="s",
        num_cores=_NC, num_subcores=_NS,
    )


def _wid():
    return lax.axis_index("c") * _NS + lax.axis_index("s")


def _any_lane(m):
    return plsc.all_reduce_population_count(m)[0] > 0


def _prep_body(p_ref, cls_ref, upd_ref):
    x = p_ref[...]
    ci = lax.broadcasted_iota(jnp.int32, x.shape, 0).astype(jnp.float32)
    cls_ref[...] = jnp.sum(x * ci, axis=0)
    upd_ref[...] = jnp.full(upd_ref.shape, -1.0, jnp.float32)


def _tc_prep(primT):
    nblk = (_N + _TCB - 1) // _TCB
    return pl.pallas_call(
        _prep_body,
        grid=(nblk,),
        in_specs=[pl.BlockSpec((_CLS, _TCB), lambda i: (0, i))],
        out_specs=[
            pl.BlockSpec((_TCB,), lambda i: (i,)),
            pl.BlockSpec((_TCB,), lambda i: (i,)),
        ],
        out_shape=[
            jax.ShapeDtypeStruct((_N,), jnp.float32),
            jax.ShapeDtypeStruct((_N,), jnp.float32),
        ],
    )(primT)


def _apply_body(p_ref, u_ref, out_ref):
    x = p_ref[...]
    u = u_ref[...]
    ub = jnp.broadcast_to(u[None, :], x.shape)
    ci = lax.broadcasted_iota(jnp.int32, x.shape, 0).astype(jnp.float32)
    out_ref[...] = jnp.where(ub < 0.0, x, (ci == ub).astype(jnp.float32))


def _tc_apply(primT, upd):
    nblk = (_N + _TCB - 1) // _TCB
    return pl.pallas_call(
        _apply_body,
        grid=(nblk,),
        in_specs=[
            pl.BlockSpec((_CLS, _TCB), lambda i: (0, i)),
            pl.BlockSpec((_TCB,), lambda i: (i,)),
        ],
        out_specs=pl.BlockSpec((_CLS, _TCB), lambda i: (0, i)),
        out_shape=jax.ShapeDtypeStruct((_CLS, _N), jnp.float32),
    )(primT, upd)


def _p1_body(pos_hbm, val_hbm, bits_hbm, bits_v, pos_v, val_v, sem_a, sem_b):
    npairs = pos_hbm.shape[0]
    nchunk = npairs // _P1_CHUNK
    wid = _wid()
    base = wid * _ROWS_W

    def _zero(i, _):
        bits_v[pl.ds(i * _L, _L)] = jnp.zeros((_L,), jnp.int32)
        return 0

    lax.fori_loop(0, _ROWS_W_PAD // _L, _zero, 0)

    def _dmas(k, sem):
        par = (k % 2) * _P1_CHUNK
        sl = pl.ds(k * _P1_CHUNK, _P1_CHUNK)
        dsl = pl.ds(par, _P1_CHUNK)
        return (pltpu.make_async_copy(pos_hbm.at[sl], pos_v.at[dsl], sem),
                pltpu.make_async_copy(val_hbm.at[sl], val_v.at[dsl], sem))

    def _start(k):
        @pl.when(k % 2 == 0)
        def _():
            for d in _dmas(k, sem_a):
                d.start()

        @pl.when(k % 2 == 1)
        def _():
            for d in _dmas(k, sem_b):
                d.start()

    def _wait(k):
        @pl.when(k % 2 == 0)
        def _():
            for d in _dmas(k, sem_a):
                d.wait()

        @pl.when(k % 2 == 1)
        def _():
            for d in _dmas(k, sem_b):
                d.wait()

    _start(0)

    def _chunk(k, _):
        @pl.when(k + 1 < nchunk)
        def _():
            _start(k + 1)

        _wait(k)
        par = (k % 2) * _P1_CHUNK

        def _group(g, _):
            off = par + g * 2 * _L
            pos_a = pos_v[pl.ds(off, _L)]
            pos_b = pos_v[pl.ds(off + _L, _L)]
            rel_a = pos_a - base
            rel_b = pos_b - base
            m_a = (rel_a >= 0) & (rel_a < _ROWS_W)
            m_b = (rel_b >= 0) & (rel_b < _ROWS_W)
            any_in = _any_lane(m_a | m_b)

            @pl.when(any_in)
            def _():
                for (rel, m, voff) in ((rel_a, m_a, off), (rel_b, m_b, off + _L)):
                    val = val_v[pl.ds(voff, _L)]
                    srel = jnp.clip(rel, 0, _ROWS_W - 1)
                    bitv = jnp.int32(1) << val

                    def _cond(pending):
                        return _any_lane(pending)

                    def _rmw(pending):
                        old = plsc.load_gather(bits_v, [srel], mask=pending)
                        plsc.store_scatter(bits_v, [srel], old | bitv, mask=pending)
                        cur = plsc.load_gather(bits_v, [srel], mask=pending)
                        return pending & ((cur & bitv) != bitv)

                    lax.while_loop(_cond, _rmw, m)

            return 0

        lax.fori_loop(0, _P1_CHUNK // (2 * _L), _group, 0)
        return 0

    lax.fori_loop(0, nchunk, _chunk, 0)
    pltpu.async_copy(bits_v.at[pl.ds(0, _ROWS_W)], bits_hbm.at[wid], sem_a).wait()


def _sc_phase1(pos1, val1):
    return pl.kernel(
        _p1_body,
        out_type=jax.ShapeDtypeStruct((_NW, _ROWS_W), jnp.int32),
        mesh=_mesh(),
        compiler_params=_SC_PARAMS,
        scratch_types=[
            pltpu.VMEM((_ROWS_W_PAD,), jnp.int32),
            pltpu.VMEM((2 * _P1_CHUNK,), jnp.int32),
            pltpu.VMEM((2 * _P1_CHUNK,), jnp.int32),
            pltpu.SemaphoreType.DMA,
            pltpu.SemaphoreType.DMA,
        ],
    )(pos1, val1)


def _p2_body(upd_ref, bits_hbm, clsp_hbm, pos2_hbm, w_hbm, b_hbm,
             p2_v, bits_b, cls_b, av, vv, w_v, b_v, wb_v, t_v,
             sem1, sem2, sem3):
    nrow = p2_v.shape[0]
    wid = _wid()
    pltpu.async_copy(pos2_hbm.at[wid], p2_v, sem1).wait()
    pltpu.async_copy(w_hbm, w_v, sem1).wait()
    pltpu.async_copy(b_hbm, b_v, sem1).wait()

    lanes = lax.iota(jnp.int32, _L)

    def _wb(g, _):
        bidx = (g % 2) * _L + lanes
        wb_v[pl.ds(g * _L, _L)] = (
            w_v[pl.ds(g * _L, _L)] + plsc.load_gather(b_v, [bidx]))
        return 0

    lax.fori_loop(0, 64, _wb, 0)

    for chunk in range(2):
        cvec = chunk * _L + lanes

        def _scan(j, carry):
            best, bi = carry
            lg = plsc.load_gather(wb_v, [cvec * 32 + j])
            take = lg > best
            return jnp.where(take, lg, best), jnp.where(take, j, bi)

        best0 = plsc.load_gather(wb_v, [cvec * 32])
        _, bi = lax.fori_loop(1, _CLS, _scan, (best0, jnp.zeros((_L,), jnp.int32)))
        t_v[pl.ds(chunk * _L, _L)] = bi

    gathers = []
    for j in range(nrow):
        gathers.append(pltpu.async_copy(bits_hbm.at[p2_v.at[j]], bits_b.at[j], sem1))
        gathers.append(pltpu.async_copy(clsp_hbm.at[p2_v.at[j]], cls_b.at[j], sem2))
    for g in gathers:
        g.wait()

    def _row(j, _):
        for l in range(_GB // _L):
            sl = pl.ds(l * _L, _L)
            pos = p2_v[j, sl]
            bits = bits_b[j, sl]
            c_old = cls_b[j, sl].astype(jnp.int32)
            is0 = bits == 0
            one_hot = (bits & (bits - 1)) == 0
            f = bits.astype(jnp.float32)
            expo = (plsc.bitcast(f, jnp.int32) >> 23) - 127
            c_one = jnp.clip(jnp.where(is0, c_old, expo), 0, _CLS - 1)
            cls = plsc.load_gather(t_v, [c_one])

            multi = ~one_hot
            any_multi = _any_lane(multi)

            def _hard():
                def _outer(j2, carry):
                    best, bi = carry

                    def _inner(v, acc):
                        on = ((bits >> v) & 1) == 1
                        wv = plsc.load_gather(
                            w_v, [jnp.broadcast_to(v * 32 + j2, (_L,))])
                        return acc + jnp.where(on, wv, 0.0)

                    lg = lax.fori_loop(0, _CLS, _inner, jnp.zeros((_L,), jnp.float32))
                    lg = lg + plsc.load_gather(b_v, [jnp.broadcast_to(j2, (_L,))])
                    take = lg > best
                    return jnp.where(take, lg, best), jnp.where(take, j2, bi)

                init = (jnp.full((_L,), -jnp.inf, jnp.float32),
                        jnp.zeros((_L,), jnp.int32))
                _, bi = lax.fori_loop(0, _CLS, _outer, init)
                return jnp.where(multi, bi, cls)

            cls = lax.cond(any_multi, _hard, lambda: cls)

            av[j, sl] = pos
            vv[j, sl] = cls.astype(jnp.float32)
        return 0

    lax.fori_loop(0, nrow, _row, 0)

    scatters = []
    for j in range(nrow):
        scatters.append(pltpu.async_copy(vv.at[j], upd_ref.at[av.at[j]], sem3))
    for s in scatters:
        s.wait()


def _sc_phase2(upd_flat_ref, bits, clsp, pos2, w, b_pad, nrow):
    pl.kernel(
        _p2_body,
        out_type=(),
        mesh=_mesh(),
        compiler_params=_SC_PARAMS,
        scratch_types=[
            pltpu.VMEM((nrow, _GB), jnp.int32),
            pltpu.VMEM((nrow, _GB), jnp.int32),
            pltpu.VMEM((nrow, _GB), jnp.float32),
            pltpu.VMEM((nrow, _GB), jnp.int32),
            pltpu.VMEM((nrow, _GB), jnp.float32),
            pltpu.VMEM((1024,), jnp.float32),
            pltpu.VMEM((32,), jnp.float32),
            pltpu.VMEM((1024,), jnp.float32),
            pltpu.VMEM((32,), jnp.int32),
            pltpu.SemaphoreType.DMA,
            pltpu.SemaphoreType.DMA,
            pltpu.SemaphoreType.DMA,
        ],
    )(upd_flat_ref, bits, clsp, pos2, w, b_pad)


def kernel(primary, W, b, positions1, values1, positions2):
    p1 = positions1.shape[0]
    p2 = positions2.shape[0]

    pad1 = (-p1) % _P1_CHUNK
    if pad1:
        positions1 = jnp.concatenate(
            [positions1, jnp.full((pad1,), -1, positions1.dtype)])
        values1 = jnp.concatenate([values1, jnp.zeros((pad1,), values1.dtype)])

    pad2 = (-p2) % (_NW * _GB)
    if pad2:
        reps = -(-pad2 // p2)
        extra = jnp.tile(positions2, reps)[:pad2]
        positions2 = jnp.concatenate([positions2, extra])
    nrow = positions2.shape[0] // (_NW * _GB)
    pos2 = positions2.reshape(_NW, nrow, _GB).astype(jnp.int32)

    b_pad = jnp.concatenate([b, jnp.zeros((32 - _CLS,), b.dtype)])
    w_pad = jnp.zeros((32, 32), W.dtype).at[:_CLS, :_CLS].set(W).reshape(1024)

    primT = primary.T
    clsp, upd0 = _tc_prep(primT)
    bits = _sc_phase1(positions1.astype(jnp.int32),
                      values1.astype(jnp.int32)).reshape(_N)

    uref = jax.new_ref(upd0)
    _sc_phase2(uref, bits, clsp, pos2, w_pad, b_pad, nrow)
    outT = _tc_apply(primT, uref[...])
    return outT.T
